# Initial kernel scaffold; baseline (speedup 1.0000x reference)
import jax
import jax.numpy as jnp
from jax.experimental import pallas as pl


def kernel(x, conv1_w, conv1_b, conv2_w, conv2_b, gcn1_w, gcn1_b, gcn2_w, gcn2_b, fc_w, fc_b):
    raise NotImplementedError("write your pallas kernel here")


# speedup vs baseline: 9.4715x; 9.4715x over previous
import jax
import jax.numpy as jnp
from jax.experimental import pallas as pl

_K = 5
_RT = 256


def _shift_rows(a, d):
    if d == 0:
        return a
    z = jnp.zeros((abs(d), a.shape[1]), a.dtype)
    if d > 0:
        return jnp.concatenate([a[d:], z], axis=0)
    return jnp.concatenate([z, a[:d]], axis=0)


def _blocked_conv_weights(w, m):
    cout, cin, kk = w.shape
    jdim = m + kk - 1
    wt = jnp.transpose(w, (2, 1, 0))
    wb = jnp.zeros((jdim, cin, m, cout), w.dtype)
    for u in range(m):
        for k in range(kk):
            wb = wb.at[u + k, :, u, :].set(wt[k])
    return wb.reshape(jdim * cin, m * cout)


def _conv_stage(xr_ref, w1_ref, b1_ref, w2_ref, b2_ref, out_ref):
    xr = xr_ref[0]
    cin = 64
    xw = jnp.concatenate(
        [_shift_rows(xr, -1)[:, 6 * cin:], xr, _shift_rows(xr, 1)[:, :2 * cin]],
        axis=1,
    )
    y = jnp.dot(xw, w1_ref[...], preferred_element_type=jnp.float32) + b1_ref[...]
    y = jax.nn.relu(y)
    h = jnp.concatenate(
        [jnp.maximum(y[:, 32 * r: 32 * r + 16], y[:, 32 * r + 16: 32 * r + 32])
         for r in range(4)],
        axis=1,
    )
    hw = jnp.concatenate(
        [_shift_rows(h, -1)[:, 32:], h, _shift_rows(h, 1)[:, :32]], axis=1
    )
    y2 = jnp.dot(hw, w2_ref[...], preferred_element_type=jnp.float32) + b2_ref[...]
    y2 = jax.nn.relu(y2)
    out_ref[0] = jnp.concatenate(
        [jnp.maximum(y2[:, 64 * s: 64 * s + 32], y2[:, 64 * s + 32: 64 * s + 64])
         for s in range(2)],
        axis=1,
    )


def _chain_coeffs(n, rt):
    t = jax.lax.broadcasted_iota(jnp.int32, (n, 1), 0) % rt
    inv2 = 1.0 / jnp.sqrt(2.0)
    inv3 = 1.0 / jnp.sqrt(3.0)
    dinv = jnp.where((t == 0) | (t == rt - 1), inv2, inv3)
    c_self = dinv * dinv
    c_prev = jnp.where(t == 0, 0.0, dinv * jnp.where(t == 1, inv2, inv3))
    c_next = jnp.where(t == rt - 1, 0.0, dinv * jnp.where(t == rt - 2, inv2, inv3))
    return c_prev, c_self, c_next


def _gcn(h, w_ref, b_ref, c_prev, c_self, c_next):
    y = jnp.dot(h, w_ref[...], preferred_element_type=jnp.float32)
    agg = c_prev * _shift_rows(y, -1) + c_self * y + c_next * _shift_rows(y, 1)
    return jax.nn.relu(agg + b_ref[...])


def _gcn_stage(h_ref, g1w_ref, g1b_ref, g2w_ref, g2b_ref, fcw_ref, fcb_ref,
               out_ref):
    h = h_ref[...]
    n = h.shape[0]
    c_prev, c_self, c_next = _chain_coeffs(n, _RT)
    h = _gcn(h, g1w_ref, g1b_ref, c_prev, c_self, c_next)
    h = _gcn(h, g2w_ref, g2b_ref, c_prev, c_self, c_next)
    g = n // _RT
    gi = jax.lax.broadcasted_iota(jnp.int32, (g, n), 0)
    ni = jax.lax.broadcasted_iota(jnp.int32, (g, n), 1)
    mean_mat = jnp.where(ni // _RT == gi, 1.0 / _RT, 0.0)
    hm = jnp.dot(mean_mat, h, preferred_element_type=jnp.float32)
    out_ref[...] = (
        jnp.dot(hm, fcw_ref[...], preferred_element_type=jnp.float32) + fcb_ref[...]
    )


def kernel(x, conv1_w, conv1_b, conv2_w, conv2_b, gcn1_w, gcn1_b,
           gcn2_w, gcn2_b, fc_w, fc_b):
    b, t, cin = x.shape
    c1 = conv1_w.shape[0]
    c2 = conv2_w.shape[0]
    out_dim = fc_w.shape[0]
    tb = t // 8

    xr = x.reshape(b, tb, 8 * cin)
    w1b = _blocked_conv_weights(conv1_w, 8)
    w2b = _blocked_conv_weights(conv2_w, 4)
    b1t = jnp.tile(conv1_b, 8).reshape(1, -1)
    b2t = jnp.tile(conv2_b, 4).reshape(1, -1)
    fcw = fc_w.T
    g1b = gcn1_b.reshape(1, -1)
    g2b = gcn2_b.reshape(1, -1)
    fcb = fc_b.reshape(1, -1)

    full = lambda a: pl.BlockSpec(a.shape, lambda i: (0,) * a.ndim)
    h = pl.pallas_call(
        _conv_stage,
        grid=(b,),
        in_specs=[
            pl.BlockSpec((1, tb, 8 * cin), lambda i: (i, 0, 0)),
            full(w1b), full(b1t), full(w2b), full(b2t),
        ],
        out_specs=pl.BlockSpec((1, tb, 2 * c2), lambda i: (i, 0, 0)),
        out_shape=jax.ShapeDtypeStruct((b, tb, 2 * c2), jnp.float32),
    )(xr, w1b, b1t, w2b, b2t)

    nodes = h.reshape(b * _RT, c2)

    chains_per_blk = min(8, b)
    rows = chains_per_blk * _RT
    out = pl.pallas_call(
        _gcn_stage,
        grid=(b // chains_per_blk,),
        in_specs=[
            pl.BlockSpec((rows, c2), lambda i: (i, 0)),
            full(gcn1_w), full(g1b), full(gcn2_w), full(g2b),
            full(fcw), full(fcb),
        ],
        out_specs=pl.BlockSpec((chains_per_blk, out_dim), lambda i: (i, 0)),
        out_shape=jax.ShapeDtypeStruct((b, out_dim), jnp.float32),
    )(nodes, gcn1_w, g1b, gcn2_w, g2b, fcw, fcb)
    return out


# speedup vs baseline: 9.5692x; 1.0103x over previous
import jax
import jax.numpy as jnp
from jax.experimental import pallas as pl

_K = 5
_RT = 256


def _shift_rows(a, d):
    if d == 0:
        return a
    z = jnp.zeros((abs(d), a.shape[1]), a.dtype)
    if d > 0:
        return jnp.concatenate([a[d:], z], axis=0)
    return jnp.concatenate([z, a[:d]], axis=0)


def _blocked_conv_weights(w, m):
    cout, cin, kk = w.shape
    jdim = m + kk - 1
    wt = jnp.transpose(w, (2, 1, 0))
    wb = jnp.zeros((jdim, cin, m, cout), w.dtype)
    for u in range(m):
        for k in range(kk):
            wb = wb.at[u + k, :, u, :].set(wt[k])
    return wb.reshape(jdim * cin, m * cout)


def _conv_stage(xr_ref, w1_ref, b1_ref, w2_ref, b2_ref, out_ref):
    xr = xr_ref[0]
    cin = 64
    xw = jnp.concatenate(
        [_shift_rows(xr, -1)[:, 6 * cin:], xr, _shift_rows(xr, 1)[:, :2 * cin]],
        axis=1,
    )
    y = jnp.dot(xw.astype(jnp.bfloat16), w1_ref[...].astype(jnp.bfloat16),
                preferred_element_type=jnp.float32) + b1_ref[...]
    y = jax.nn.relu(y)
    h = jnp.concatenate(
        [jnp.maximum(y[:, 32 * r: 32 * r + 16], y[:, 32 * r + 16: 32 * r + 32])
         for r in range(4)],
        axis=1,
    )
    hw = jnp.concatenate(
        [_shift_rows(h, -1)[:, 32:], h, _shift_rows(h, 1)[:, :32]], axis=1
    )
    y2 = jnp.dot(hw.astype(jnp.bfloat16), w2_ref[...].astype(jnp.bfloat16),
                 preferred_element_type=jnp.float32) + b2_ref[...]
    y2 = jax.nn.relu(y2)
    out_ref[0] = jnp.concatenate(
        [jnp.maximum(y2[:, 64 * s: 64 * s + 32], y2[:, 64 * s + 32: 64 * s + 64])
         for s in range(2)],
        axis=1,
    )


def _chain_coeffs(n, rt):
    t = jax.lax.broadcasted_iota(jnp.int32, (n, 1), 0) % rt
    inv2 = 1.0 / jnp.sqrt(2.0)
    inv3 = 1.0 / jnp.sqrt(3.0)
    dinv = jnp.where((t == 0) | (t == rt - 1), inv2, inv3)
    c_self = dinv * dinv
    c_prev = jnp.where(t == 0, 0.0, dinv * jnp.where(t == 1, inv2, inv3))
    c_next = jnp.where(t == rt - 1, 0.0, dinv * jnp.where(t == rt - 2, inv2, inv3))
    return c_prev, c_self, c_next


def _gcn(h, w_ref, b_ref, c_prev, c_self, c_next):
    y = jnp.dot(h.astype(jnp.bfloat16), w_ref[...].astype(jnp.bfloat16),
                preferred_element_type=jnp.float32)
    agg = c_prev * _shift_rows(y, -1) + c_self * y + c_next * _shift_rows(y, 1)
    return jax.nn.relu(agg + b_ref[...])


def _gcn_stage(h_ref, g1w_ref, g1b_ref, g2w_ref, g2b_ref, fcw_ref, fcb_ref,
               out_ref):
    h = h_ref[...]
    n = h.shape[0]
    c_prev, c_self, c_next = _chain_coeffs(n, _RT)
    h = _gcn(h, g1w_ref, g1b_ref, c_prev, c_self, c_next)
    h = _gcn(h, g2w_ref, g2b_ref, c_prev, c_self, c_next)
    g = n // _RT
    gi = jax.lax.broadcasted_iota(jnp.int32, (g, n), 0)
    ni = jax.lax.broadcasted_iota(jnp.int32, (g, n), 1)
    mean_mat = jnp.where(ni // _RT == gi, 1.0 / _RT, 0.0)
    hm = jnp.dot(mean_mat, h, preferred_element_type=jnp.float32)
    out_ref[...] = (
        jnp.dot(hm, fcw_ref[...], preferred_element_type=jnp.float32) + fcb_ref[...]
    )


def kernel(x, conv1_w, conv1_b, conv2_w, conv2_b, gcn1_w, gcn1_b,
           gcn2_w, gcn2_b, fc_w, fc_b):
    b, t, cin = x.shape
    c1 = conv1_w.shape[0]
    c2 = conv2_w.shape[0]
    out_dim = fc_w.shape[0]
    tb = t // 8

    xr = x.reshape(b, tb, 8 * cin)
    w1b = _blocked_conv_weights(conv1_w, 8)
    w2b = _blocked_conv_weights(conv2_w, 4)
    b1t = jnp.tile(conv1_b, 8).reshape(1, -1)
    b2t = jnp.tile(conv2_b, 4).reshape(1, -1)
    fcw = fc_w.T
    g1b = gcn1_b.reshape(1, -1)
    g2b = gcn2_b.reshape(1, -1)
    fcb = fc_b.reshape(1, -1)

    full = lambda a: pl.BlockSpec(a.shape, lambda i: (0,) * a.ndim)
    h = pl.pallas_call(
        _conv_stage,
        grid=(b,),
        in_specs=[
            pl.BlockSpec((1, tb, 8 * cin), lambda i: (i, 0, 0)),
            full(w1b), full(b1t), full(w2b), full(b2t),
        ],
        out_specs=pl.BlockSpec((1, tb, 2 * c2), lambda i: (i, 0, 0)),
        out_shape=jax.ShapeDtypeStruct((b, tb, 2 * c2), jnp.float32),
    )(xr, w1b, b1t, w2b, b2t)

    nodes = h.reshape(b * _RT, c2)

    chains_per_blk = min(8, b)
    rows = chains_per_blk * _RT
    out = pl.pallas_call(
        _gcn_stage,
        grid=(b // chains_per_blk,),
        in_specs=[
            pl.BlockSpec((rows, c2), lambda i: (i, 0)),
            full(gcn1_w), full(g1b), full(gcn2_w), full(g2b),
            full(fcw), full(fcb),
        ],
        out_specs=pl.BlockSpec((chains_per_blk, out_dim), lambda i: (i, 0)),
        out_shape=jax.ShapeDtypeStruct((b, out_dim), jnp.float32),
    )(nodes, gcn1_w, g1b, gcn2_w, g2b, fcw, fcb)
    return out


# speedup vs baseline: 12.1814x; 1.2730x over previous
import jax
import jax.numpy as jnp
from jax.experimental import pallas as pl

_K = 5
_RT = 256
_BA = 8
_CB = 8


def _shift_rows(a, d):
    if d == 0:
        return a
    z = jnp.zeros((abs(d),) + a.shape[1:], a.dtype)
    if d > 0:
        return jnp.concatenate([a[d:], z], axis=0)
    return jnp.concatenate([z, a[:d]], axis=0)


def _blocked_conv_weights(w, m):
    cout, cin, kk = w.shape
    jdim = m + kk - 1
    wt = jnp.transpose(w, (2, 1, 0))
    wb = jnp.zeros((jdim, cin, m, cout), w.dtype)
    for u in range(m):
        for k in range(kk):
            wb = wb.at[u + k, :, u, :].set(wt[k])
    return wb.reshape(jdim * cin, m * cout)


def _chain_stencil_matrix(rt):
    t = jnp.arange(rt)
    deg = jnp.where((t == 0) | (t == rt - 1), 2.0, 3.0)
    dinv = jax.lax.rsqrt(deg)
    band = jnp.abs(t[:, None] - t[None, :]) <= 1
    return jnp.where(band, dinv[:, None] * dinv[None, :], 0.0)


def _edge_mask(rows, period, lo):
    r = jax.lax.broadcasted_iota(jnp.int32, (rows, 1), 0) % period
    bad = (r == 0) if lo else (r == period - 1)
    return jnp.where(bad, 0.0, 1.0).astype(jnp.bfloat16)


def _conv_stage(xr_ref, w1_ref, b1_ref, w2_ref, b2_ref, out_ref):
    xr = xr_ref[...].astype(jnp.bfloat16)
    rows = xr.shape[0]
    cin = 64
    m_lo = _edge_mask(rows, 128, True)
    m_hi = _edge_mask(rows, 128, False)
    xw = jnp.concatenate(
        [_shift_rows(xr, -1)[:, 6 * cin:] * m_lo, xr,
         _shift_rows(xr, 1)[:, :2 * cin] * m_hi],
        axis=1,
    )
    y = jnp.dot(xw, w1_ref[...], preferred_element_type=jnp.float32) + b1_ref[...]
    y = jax.nn.relu(y)
    h = jnp.concatenate(
        [jnp.maximum(y[:, 32 * r: 32 * r + 16], y[:, 32 * r + 16: 32 * r + 32])
         for r in range(4)],
        axis=1,
    ).astype(jnp.bfloat16)
    hw = jnp.concatenate(
        [_shift_rows(h, -1)[:, 32:] * m_lo, h, _shift_rows(h, 1)[:, :32] * m_hi],
        axis=1,
    )
    y2 = jnp.dot(hw, w2_ref[...], preferred_element_type=jnp.float32) + b2_ref[...]
    y2 = jax.nn.relu(y2)
    out_ref[...] = jnp.concatenate(
        [jnp.maximum(y2[:, 64 * s: 64 * s + 32], y2[:, 64 * s + 32: 64 * s + 64])
         for s in range(2)],
        axis=1,
    )


def _gcn_stage(h_ref, s_ref, g1w_ref, g1b_ref, g2w_ref, g2b_ref, fcw_ref,
               fcb_ref, out_ref):
    h = h_ref[...].astype(jnp.bfloat16)
    smat = s_ref[...]
    cb = h.shape[0] // _RT
    sh = jnp.concatenate(
        [jnp.dot(smat, h[_RT * c: _RT * (c + 1)],
                 preferred_element_type=jnp.float32).astype(jnp.bfloat16)
         for c in range(cb)],
        axis=0,
    )
    y1 = jax.nn.relu(
        jnp.dot(sh, g1w_ref[...], preferred_element_type=jnp.float32)
        + g1b_ref[...]
    ).astype(jnp.bfloat16)
    ones_row = jnp.full((1, _RT), 1.0 / _RT, jnp.bfloat16)
    means = []
    for c in range(cb):
        s2 = jnp.dot(smat, y1[_RT * c: _RT * (c + 1)],
                     preferred_element_type=jnp.float32).astype(jnp.bfloat16)
        y2 = jax.nn.relu(
            jnp.dot(s2, g2w_ref[...], preferred_element_type=jnp.float32)
            + g2b_ref[...]
        ).astype(jnp.bfloat16)
        means.append(jnp.dot(ones_row, y2, preferred_element_type=jnp.float32))
    hm = jnp.concatenate(means, axis=0).astype(jnp.bfloat16)
    out_ref[...] = (
        jnp.dot(hm, fcw_ref[...], preferred_element_type=jnp.float32)
        + fcb_ref[...]
    )


def kernel(x, conv1_w, conv1_b, conv2_w, conv2_b, gcn1_w, gcn1_b,
           gcn2_w, gcn2_b, fc_w, fc_b):
    b, t, cin = x.shape
    c2 = conv2_w.shape[0]
    out_dim = fc_w.shape[0]
    tb = t // 8
    bf = jnp.bfloat16

    xr = x.reshape(b * tb, 8 * cin)
    w1b = _blocked_conv_weights(conv1_w, 8).astype(bf)
    w2b = _blocked_conv_weights(conv2_w, 4).astype(bf)
    b1t = jnp.tile(conv1_b, 8).reshape(1, -1)
    b2t = jnp.tile(conv2_b, 4).reshape(1, -1)
    smat = _chain_stencil_matrix(_RT).astype(bf)
    g1w = gcn1_w.astype(bf)
    g2w = gcn2_w.astype(bf)
    fcw = fc_w.T.astype(bf)
    g1b = gcn1_b.reshape(1, -1)
    g2b = gcn2_b.reshape(1, -1)
    fcb = fc_b.reshape(1, -1)

    full = lambda a: pl.BlockSpec(a.shape, lambda i: (0,) * a.ndim)
    ba = min(_BA, b)
    cb = min(_CB, b)
    arows = ba * tb
    h = pl.pallas_call(
        _conv_stage,
        grid=(b // ba,),
        in_specs=[
            pl.BlockSpec((arows, 8 * cin), lambda i: (i, 0)),
            full(w1b), full(b1t), full(w2b), full(b2t),
        ],
        out_specs=pl.BlockSpec((arows, 2 * c2), lambda i: (i, 0)),
        out_shape=jax.ShapeDtypeStruct((b * tb, 2 * c2), jnp.float32),
    )(xr, w1b, b1t, w2b, b2t)

    nodes = h.reshape(b * 2 * tb, c2)

    rows = cb * _RT
    out = pl.pallas_call(
        _gcn_stage,
        grid=(b // cb,),
        in_specs=[
            pl.BlockSpec((rows, c2), lambda i: (i, 0)),
            full(smat), full(g1w), full(g1b), full(g2w), full(g2b),
            full(fcw), full(fcb),
        ],
        out_specs=pl.BlockSpec((cb, out_dim), lambda i: (i, 0)),
        out_shape=jax.ShapeDtypeStruct((b, out_dim), jnp.float32),
    )(nodes, smat, g1w, g1b, g2w, g2b, fcw, fcb)
    return out


# speedup vs baseline: 14.2108x; 1.1666x over previous
import jax
import jax.numpy as jnp
import numpy as np
from jax.experimental import pallas as pl

_K = 5
_RT = 256
_BA = 8
_CB = 8


def _shift_rows(a, d):
    if d == 0:
        return a
    z = jnp.zeros((abs(d),) + a.shape[1:], a.dtype)
    if d > 0:
        return jnp.concatenate([a[d:], z], axis=0)
    return jnp.concatenate([z, a[:d]], axis=0)


def _blocked_conv_weights(w, m):
    cout, cin, kk = w.shape
    jdim = m + kk - 1
    sel = np.zeros((jdim, m, kk), np.float32)
    for u in range(m):
        for k in range(kk):
            sel[u + k, u, k] = 1.0
    wb = jnp.einsum("juk,cik->jiuc", sel, w)
    return wb.reshape(jdim * cin, m * cout)


def _chain_stencil_matrix(rt):
    t = np.arange(rt)
    deg = np.where((t == 0) | (t == rt - 1), 2.0, 3.0)
    dinv = 1.0 / np.sqrt(deg)
    band = np.abs(t[:, None] - t[None, :]) <= 1
    return np.where(band, dinv[:, None] * dinv[None, :], 0.0).astype(np.float32)


def _mean_matrix(groups, rt):
    m = np.zeros((groups, groups * rt), np.float32)
    for g in range(groups):
        m[g, g * rt:(g + 1) * rt] = 1.0 / rt
    return m


def _edge_mask(rows, period, lo):
    r = jax.lax.broadcasted_iota(jnp.int32, (rows, 1), 0) % period
    bad = (r == 0) if lo else (r == period - 1)
    return jnp.where(bad, 0.0, 1.0).astype(jnp.bfloat16)


def _conv_stage(xr_ref, w1_ref, b1_ref, w2_ref, b2_ref, out_ref):
    xr = xr_ref[...].astype(jnp.bfloat16)
    rows = xr.shape[0]
    cin = 64
    m_lo = _edge_mask(rows, 128, True)
    m_hi = _edge_mask(rows, 128, False)
    xw = jnp.concatenate(
        [_shift_rows(xr, -1)[:, 6 * cin:] * m_lo, xr,
         _shift_rows(xr, 1)[:, :2 * cin] * m_hi],
        axis=1,
    )
    y = jnp.dot(xw, w1_ref[...], preferred_element_type=jnp.float32) + b1_ref[...]
    y = jax.nn.relu(y)
    h = jnp.concatenate(
        [jnp.maximum(y[:, 32 * r: 32 * r + 16], y[:, 32 * r + 16: 32 * r + 32])
         for r in range(4)],
        axis=1,
    ).astype(jnp.bfloat16)
    hw = jnp.concatenate(
        [_shift_rows(h, -1)[:, 32:] * m_lo, h, _shift_rows(h, 1)[:, :32] * m_hi],
        axis=1,
    )
    y2 = jnp.dot(hw, w2_ref[...], preferred_element_type=jnp.float32) + b2_ref[...]
    y2 = jax.nn.relu(y2)
    out_ref[...] = jnp.concatenate(
        [jnp.maximum(y2[:, 64 * s: 64 * s + 32], y2[:, 64 * s + 32: 64 * s + 64])
         for s in range(2)],
        axis=1,
    )


def _chain_coeffs(n, rt):
    t = jax.lax.broadcasted_iota(jnp.int32, (n, 1), 0) % rt
    inv2 = 1.0 / np.sqrt(2.0)
    inv3 = 1.0 / np.sqrt(3.0)
    dinv = jnp.where((t == 0) | (t == rt - 1), inv2, inv3)
    c_self = dinv * dinv
    c_prev = jnp.where(t == 0, 0.0, dinv * jnp.where(t == 1, inv2, inv3))
    c_next = jnp.where(t == rt - 1, 0.0, dinv * jnp.where(t == rt - 2, inv2, inv3))
    return c_prev, c_self, c_next


def _gcn_stage(h_ref, s_ref, mm_ref, g1w_ref, g1b_ref, g2w_ref, g2b_ref,
               fcw_ref, fcb_ref, out_ref):
    h = h_ref[...]
    n = h.shape[0]
    cb = n // _RT
    c_prev, c_self, c_next = _chain_coeffs(n, _RT)
    sh = (c_prev * _shift_rows(h, -1) + c_self * h
          + c_next * _shift_rows(h, 1)).astype(jnp.bfloat16)
    y1 = jax.nn.relu(
        jnp.dot(sh, g1w_ref[...], preferred_element_type=jnp.float32)
        + g1b_ref[...]
    ).astype(jnp.bfloat16)
    smat = s_ref[...]
    s2 = jnp.concatenate(
        [jnp.dot(smat, y1[_RT * c: _RT * (c + 1)],
                 preferred_element_type=jnp.float32).astype(jnp.bfloat16)
         for c in range(cb)],
        axis=0,
    )
    y2 = jax.nn.relu(
        jnp.dot(s2, g2w_ref[...], preferred_element_type=jnp.float32)
        + g2b_ref[...]
    ).astype(jnp.bfloat16)
    hm = jnp.dot(mm_ref[...], y2,
                 preferred_element_type=jnp.float32).astype(jnp.bfloat16)
    out_ref[...] = (
        jnp.dot(hm, fcw_ref[...], preferred_element_type=jnp.float32)
        + fcb_ref[...]
    )


def kernel(x, conv1_w, conv1_b, conv2_w, conv2_b, gcn1_w, gcn1_b,
           gcn2_w, gcn2_b, fc_w, fc_b):
    b, t, cin = x.shape
    c2 = conv2_w.shape[0]
    out_dim = fc_w.shape[0]
    tb = t // 8
    bf = jnp.bfloat16

    xr = x.reshape(b * tb, 8 * cin)
    w1b = _blocked_conv_weights(conv1_w, 8).astype(bf)
    w2b = _blocked_conv_weights(conv2_w, 4).astype(bf)
    b1t = jnp.tile(conv1_b, 8).reshape(1, -1)
    b2t = jnp.tile(conv2_b, 4).reshape(1, -1)
    cb = min(_CB, b)
    smat = jnp.asarray(_chain_stencil_matrix(_RT), dtype=bf)
    mmat = jnp.asarray(_mean_matrix(cb, _RT), dtype=bf)
    g1w = gcn1_w.astype(bf)
    g2w = gcn2_w.astype(bf)
    fcw = fc_w.T.astype(bf)
    g1b = gcn1_b.reshape(1, -1)
    g2b = gcn2_b.reshape(1, -1)
    fcb = fc_b.reshape(1, -1)

    full = lambda a: pl.BlockSpec(a.shape, lambda i: (0,) * a.ndim)
    ba = min(_BA, b)
    arows = ba * tb
    h = pl.pallas_call(
        _conv_stage,
        grid=(b // ba,),
        in_specs=[
            pl.BlockSpec((arows, 8 * cin), lambda i: (i, 0)),
            full(w1b), full(b1t), full(w2b), full(b2t),
        ],
        out_specs=pl.BlockSpec((arows, 2 * c2), lambda i: (i, 0)),
        out_shape=jax.ShapeDtypeStruct((b * tb, 2 * c2), jnp.float32),
    )(xr, w1b, b1t, w2b, b2t)

    nodes = h.reshape(b * 2 * tb, c2)

    rows = cb * _RT
    out = pl.pallas_call(
        _gcn_stage,
        grid=(b // cb,),
        in_specs=[
            pl.BlockSpec((rows, c2), lambda i: (i, 0)),
            full(smat), full(mmat), full(g1w), full(g1b), full(g2w), full(g2b),
            full(fcw), full(fcb),
        ],
        out_specs=pl.BlockSpec((cb, out_dim), lambda i: (i, 0)),
        out_shape=jax.ShapeDtypeStruct((b, out_dim), jnp.float32),
    )(nodes, smat, mmat, g1w, g1b, g2w, g2b, fcw, fcb)
    return out


# speedup vs baseline: 18.0328x; 1.2689x over previous
import jax
import jax.numpy as jnp
import numpy as np
from jax.experimental import pallas as pl

_K = 5
_RT = 256
_BA = 8
_CB = 8


def _shift_rows(a, d):
    if d == 0:
        return a
    z = jnp.zeros((abs(d),) + a.shape[1:], a.dtype)
    if d > 0:
        return jnp.concatenate([a[d:], z], axis=0)
    return jnp.concatenate([z, a[:d]], axis=0)


def _blocked_conv_weights(w, m):
    cout, cin, kk = w.shape
    jdim = m + kk - 1
    sel = np.zeros((jdim, m, kk), np.float32)
    for u in range(m):
        for k in range(kk):
            sel[u + k, u, k] = 1.0
    wb = jnp.einsum("juk,cik->jiuc", sel, w)
    return wb.reshape(jdim * cin, m * cout)


def _chain_stencil_matrix(rt):
    t = np.arange(rt)
    deg = np.where((t == 0) | (t == rt - 1), 2.0, 3.0)
    dinv = 1.0 / np.sqrt(deg)
    band = np.abs(t[:, None] - t[None, :]) <= 1
    return np.where(band, dinv[:, None] * dinv[None, :], 0.0).astype(np.float32)


def _mean_matrix(groups, rt):
    m = np.zeros((groups, groups * rt), np.float32)
    for g in range(groups):
        m[g, g * rt:(g + 1) * rt] = 1.0 / rt
    return m


def _edge_mask(rows, period, lo):
    r = jax.lax.broadcasted_iota(jnp.int32, (rows, 1), 0) % period
    bad = (r == 0) if lo else (r == period - 1)
    return jnp.where(bad, 0.0, 1.0).astype(jnp.bfloat16)


def _conv_stage(x4_ref, w1_ref, b1_ref, w2_ref, b2_ref, out_ref):
    x4 = x4_ref[...].astype(jnp.bfloat16)
    ba, tbr, m, cin = x4.shape
    xr = jnp.concatenate(
        [jnp.concatenate([x4[c, :, r, :] for r in range(m)], axis=1)
         for c in range(ba)],
        axis=0,
    )
    rows = xr.shape[0]
    m_lo = _edge_mask(rows, tbr, True)
    m_hi = _edge_mask(rows, tbr, False)
    xw = jnp.concatenate(
        [_shift_rows(xr, -1)[:, 6 * cin:] * m_lo, xr,
         _shift_rows(xr, 1)[:, :2 * cin] * m_hi],
        axis=1,
    )
    y = jnp.dot(xw, w1_ref[...], preferred_element_type=jnp.float32) + b1_ref[...]
    y = jax.nn.relu(y)
    h = jnp.concatenate(
        [jnp.maximum(y[:, 32 * r: 32 * r + 16], y[:, 32 * r + 16: 32 * r + 32])
         for r in range(4)],
        axis=1,
    ).astype(jnp.bfloat16)
    hw = jnp.concatenate(
        [_shift_rows(h, -1)[:, 32:] * m_lo, h, _shift_rows(h, 1)[:, :32] * m_hi],
        axis=1,
    )
    y2 = jnp.dot(hw, w2_ref[...], preferred_element_type=jnp.float32) + b2_ref[...]
    y2 = jax.nn.relu(y2)
    out_ref[...] = jnp.concatenate(
        [jnp.maximum(y2[:, 64 * s: 64 * s + 32], y2[:, 64 * s + 32: 64 * s + 64])
         for s in range(2)],
        axis=1,
    )


def _gcn_stage(hp_ref, s0_ref, s1_ref, s_ref, mm_ref, g1w_ref, g1b_ref,
               g2w_ref, g2b_ref, fcw_ref, fcb_ref, out_ref):
    hp = hp_ref[...].astype(jnp.bfloat16)
    c2 = hp.shape[1] // 2
    half = hp.shape[0] // _CB
    s0 = s0_ref[...]
    s1 = s1_ref[...]
    sh = jnp.concatenate(
        [(jnp.dot(s0, hp[half * c: half * (c + 1), :c2],
                  preferred_element_type=jnp.float32)
          + jnp.dot(s1, hp[half * c: half * (c + 1), c2:],
                    preferred_element_type=jnp.float32)).astype(jnp.bfloat16)
         for c in range(_CB)],
        axis=0,
    )
    y1 = jax.nn.relu(
        jnp.dot(sh, g1w_ref[...], preferred_element_type=jnp.float32)
        + g1b_ref[...]
    ).astype(jnp.bfloat16)
    smat = s_ref[...]
    s2 = jnp.concatenate(
        [jnp.dot(smat, y1[_RT * c: _RT * (c + 1)],
                 preferred_element_type=jnp.float32).astype(jnp.bfloat16)
         for c in range(_CB)],
        axis=0,
    )
    y2 = jax.nn.relu(
        jnp.dot(s2, g2w_ref[...], preferred_element_type=jnp.float32)
        + g2b_ref[...]
    ).astype(jnp.bfloat16)
    hm = jnp.dot(mm_ref[...], y2,
                 preferred_element_type=jnp.float32).astype(jnp.bfloat16)
    out_ref[...] = (
        jnp.dot(hm, fcw_ref[...], preferred_element_type=jnp.float32)
        + fcb_ref[...]
    )


def kernel(x, conv1_w, conv1_b, conv2_w, conv2_b, gcn1_w, gcn1_b,
           gcn2_w, gcn2_b, fc_w, fc_b):
    b, t, cin = x.shape
    c2 = conv2_w.shape[0]
    out_dim = fc_w.shape[0]
    tb = t // 8
    bf = jnp.bfloat16

    x4 = x.reshape(b, tb, 8, cin)
    w1b = _blocked_conv_weights(conv1_w, 8).astype(bf)
    w2b = _blocked_conv_weights(conv2_w, 4).astype(bf)
    b1t = jnp.tile(conv1_b, 8).reshape(1, -1)
    b2t = jnp.tile(conv2_b, 4).reshape(1, -1)
    smat_np = _chain_stencil_matrix(_RT)
    smat = jnp.asarray(smat_np, dtype=bf)
    s0 = jnp.asarray(np.ascontiguousarray(smat_np[:, 0::2]), dtype=bf)
    s1 = jnp.asarray(np.ascontiguousarray(smat_np[:, 1::2]), dtype=bf)
    mmat = jnp.asarray(_mean_matrix(_CB, _RT), dtype=bf)
    g1w = gcn1_w.astype(bf)
    g2w = gcn2_w.astype(bf)
    fcw = fc_w.T.astype(bf)
    g1b = gcn1_b.reshape(1, -1)
    g2b = gcn2_b.reshape(1, -1)
    fcb = fc_b.reshape(1, -1)

    full = lambda a: pl.BlockSpec(a.shape, lambda i: (0,) * a.ndim)
    arows = _BA * tb
    h = pl.pallas_call(
        _conv_stage,
        grid=(b // _BA,),
        in_specs=[
            pl.BlockSpec((_BA, tb, 8, cin), lambda i: (i, 0, 0, 0)),
            full(w1b), full(b1t), full(w2b), full(b2t),
        ],
        out_specs=pl.BlockSpec((arows, 2 * c2), lambda i: (i, 0)),
        out_shape=jax.ShapeDtypeStruct((b * tb, 2 * c2), jnp.float32),
    )(x4, w1b, b1t, w2b, b2t)

    rows = _CB * tb
    out = pl.pallas_call(
        _gcn_stage,
        grid=(b // _CB,),
        in_specs=[
            pl.BlockSpec((rows, 2 * c2), lambda i: (i, 0)),
            full(s0), full(s1), full(smat), full(mmat),
            full(g1w), full(g1b), full(g2w), full(g2b),
            full(fcw), full(fcb),
        ],
        out_specs=pl.BlockSpec((_CB, out_dim), lambda i: (i, 0)),
        out_shape=jax.ShapeDtypeStruct((b, out_dim), jnp.float32),
    )(h, s0, s1, smat, mmat, g1w, g1b, g2w, g2b, fcw, fcb)
    return out
